# Initial kernel scaffold; baseline (speedup 1.0000x reference)
import jax
import jax.numpy as jnp
from jax.experimental import pallas as pl


def kernel(D):
    raise NotImplementedError("write your pallas kernel here")


# speedup vs baseline: 69.6250x; 69.6250x over previous
import functools

import jax
import jax.numpy as jnp
from jax import lax
from jax.experimental import pallas as pl
from jax.experimental.pallas import tpu as pltpu

_B = 8
_N = 128
_ND = 2 * _N - 1
_BIG = 10000000000.0
_GAMMA = 1.0


def _shift_right_lanes(x, fill):
    col = jnp.broadcast_to(jnp.asarray(fill, x.dtype), (_B, 1))
    return jnp.concatenate([col, x[:, :-1]], axis=1)


def _shift_left_lanes(x):
    zero = jnp.zeros((_B, 1), x.dtype)
    return jnp.concatenate([x[:, 1:], zero], axis=1)


def _dtw_kernel(dt_ref, out_ref, tsk_ref, qa_ref, qb_ref, qc_ref, esk_ref):
    f32 = jnp.float32

    cur = dt_ref[...]
    lane3 = lax.broadcasted_iota(jnp.int32, (_N, _B, _N), 2)
    for t in range(7):
        s = 1 << t
        rolled = jnp.concatenate([cur[_N - s :], cur[: _N - s]], axis=0)
        cur = jnp.where((lane3 >> t) & 1 == 1, rolled, cur)
    tsk_ref[...] = cur

    lane2 = lax.broadcasted_iota(jnp.int32, (_B, _N), 1)

    def fwd_body(k, carry):
        d1, d2 = carry
        r = jnp.where(k < _N, k, k - _N)
        t = tsk_ref[pl.ds(r, 1)][0]
        left = _shift_right_lanes(d1, _BIG)
        up = d1
        diag_fill = jnp.where(k == 0, f32(0.0), f32(_BIG))
        diag = _shift_right_lanes(d2, diag_fill)
        vmin = jnp.minimum(jnp.minimum(left, diag), up)
        wa = jnp.exp((vmin - left) / _GAMMA)
        wb = jnp.exp((vmin - diag) / _GAMMA)
        wc = jnp.exp((vmin - up) / _GAMMA)
        z = wa + wb + wc
        v = t + vmin - _GAMMA * jnp.log(z)
        v = jnp.where(lane2 > k, _BIG, v)
        valid = (lane2 <= k) & (lane2 >= k - (_N - 1))
        rz = jnp.where(valid, 1.0 / z, f32(0.0))
        qa_ref[pl.ds(k, 1)] = (wa * rz)[None]
        qb_ref[pl.ds(k, 1)] = (wb * rz)[None]
        qc_ref[pl.ds(k, 1)] = (wc * rz)[None]
        return (v, d1)

    big = jnp.full((_B, _N), _BIG, f32)
    lax.fori_loop(0, _ND, fwd_body, (big, big), unroll=2)

    zeros_row = jnp.zeros((1, _B, _N), f32)
    qb_ref[pl.ds(_ND, 1)] = zeros_row

    e_seed = jnp.where(lane2 == _N - 1, f32(1.0), f32(0.0))
    esk_ref[pl.ds(_ND - 1, 1)] = e_seed[None]

    def bwd_body(s, carry):
        e1, e2 = carry
        k = _ND - 2 - s
        qa = qa_ref[pl.ds(k + 1, 1)][0]
        qc = qc_ref[pl.ds(k + 1, 1)][0]
        qb = qb_ref[pl.ds(k + 2, 1)][0]
        e = (
            _shift_left_lanes(qa) * _shift_left_lanes(e1)
            + _shift_left_lanes(qb) * _shift_left_lanes(e2)
            + qc * e1
        )
        valid = (lane2 <= k) & (lane2 >= k - (_N - 1))
        e = jnp.where(valid, e, f32(0.0))
        esk_ref[pl.ds(k, 1)] = e[None]
        return (e, e1)

    e2_init = jnp.zeros((_B, _N), f32)
    lax.fori_loop(0, _ND - 1, bwd_body, (e_seed, e2_init), unroll=2)

    cur = esk_ref[...]
    lane3b = lax.broadcasted_iota(jnp.int32, (2 * _N, _B, _N), 2)
    for t in range(7):
        s = 1 << t
        rolled = jnp.concatenate([cur[s:], cur[:s]], axis=0)
        cur = jnp.where((lane3b >> t) & 1 == 1, rolled, cur)
    out_ref[...] = jnp.mean(cur[:_N], axis=1)


@functools.partial(jax.jit, static_argnames=())
def _dtw_batch(D):
    Dt = jnp.transpose(D, (1, 0, 2))
    return pl.pallas_call(
        _dtw_kernel,
        out_shape=jax.ShapeDtypeStruct((_N, _N), jnp.float32),
        in_specs=[pl.BlockSpec(memory_space=pltpu.VMEM)],
        out_specs=pl.BlockSpec(memory_space=pltpu.VMEM),
        scratch_shapes=[
            pltpu.VMEM((_N, _B, _N), jnp.float32),
            pltpu.VMEM((2 * _N, _B, _N), jnp.float32),
            pltpu.VMEM((2 * _N, _B, _N), jnp.float32),
            pltpu.VMEM((2 * _N, _B, _N), jnp.float32),
            pltpu.VMEM((2 * _N, _B, _N), jnp.float32),
        ],
    )(Dt)


def kernel(D):
    return _dtw_batch(D)


# speedup vs baseline: 153.0151x; 2.1977x over previous
import functools

import jax
import jax.numpy as jnp
from jax import lax
from jax.experimental import pallas as pl
from jax.experimental.pallas import tpu as pltpu

_B = 8
_N = 128
_ND = 2 * _N - 1
_BIG = 10000000000.0
_LOG2E = 1.4426950408889634
_A = 4


def _rot_r(x, s):
    return jnp.concatenate([x[:, -s:], x[:, :-s]], axis=1)


def _rot_l(x, s):
    return jnp.concatenate([x[:, s:], x[:, :s]], axis=1)


def _dtw_kernel(dt_ref, out_ref, tsk0_ref, tsk1_ref, tsk2_ref, tsk3_ref,
                qas_refs0, qas_refs1, qas_refs2, qas_refs3,
                qbs_refs0, qbs_refs1, qbs_refs2, qbs_refs3,
                qc_refs0, qc_refs1, qc_refs2, qc_refs3, esk_ref):
    f32 = jnp.float32
    qas_refs = (qas_refs0, qas_refs1, qas_refs2, qas_refs3)
    qbs_refs = (qbs_refs0, qbs_refs1, qbs_refs2, qbs_refs3)
    qc_refs = (qc_refs0, qc_refs1, qc_refs2, qc_refs3)
    tsk_refs = (tsk0_ref, tsk1_ref, tsk2_ref, tsk3_ref)

    cur = dt_ref[...] * f32(_LOG2E)
    lane3 = lax.broadcasted_iota(jnp.int32, (_N, _B, _N), 2)
    for t in range(7):
        s = 1 << t
        rolled = jnp.concatenate([cur[_N - s :], cur[: _N - s]], axis=0)
        cur = jnp.where((lane3 >> t) & 1 == 1, rolled, cur)
    tsk0_ref[...] = cur
    for a in range(1, _A):
        tsk_refs[a][...] = jnp.concatenate(
            [cur[:, :, -a:], cur[:, :, :-a]], axis=2)

    lane2 = lax.broadcasted_iota(jnp.int32, (_B, _N), 1)

    t0row = tsk0_ref[pl.ds(0, 1)][0]
    t00 = jnp.broadcast_to(t0row[:, 0:1], (_B, _N))
    p1 = tuple(
        jnp.where(lane2 == a, t00, f32(_BIG)) for a in range(_A + 1))
    p2 = tuple(jnp.full((_B, _N), _BIG, f32) for _ in range(_A))

    def fwd_body(k, carry):
        p1, p2, kmj = carry
        r = jnp.where(k < _N, k, k - _N)
        ts = [ref[pl.ds(r, 1)][0] for ref in tsk_refs]
        vmin0 = jnp.minimum(jnp.minimum(p1[1], p2[0]), p1[0])
        wa = jnp.exp2(vmin0 - p1[1])
        wb = jnp.exp2(vmin0 - p2[0])
        wc = jnp.exp2(vmin0 - p1[0])
        z = (wa + wb) + wc
        v0 = (ts[0] + vmin0) - jnp.log2(z)
        vs = [v0]
        for a in range(1, _A):
            vmin = jnp.minimum(jnp.minimum(p1[a + 1], p2[a]), p1[a])
            za = (jnp.exp2(vmin - p1[a + 1]) + jnp.exp2(vmin - p2[a])
                  + jnp.exp2(vmin - p1[a]))
            vs.append((ts[a] + vmin) - jnp.log2(za))
        v_top = jnp.where(lane2 < _A, f32(_BIG), _rot_r(v0, _A))
        vs.append(v_top)
        valid = (kmj >= 0) & (kmj <= _N - 1)
        rz = jnp.where(valid, 1.0 / z, f32(0.0))
        qa0 = wa * rz
        qb0 = wb * rz
        qc0 = wc * rz
        for a in range(_A):
            qas_refs[a][pl.ds(k, 1)] = jnp.where(
                lane2 < _N - (a + 1), _rot_l(qa0, a + 1), f32(0.0))[None]
            qbs_refs[a][pl.ds(k, 1)] = jnp.where(
                lane2 < _N - (a + 1), _rot_l(qb0, a + 1), f32(0.0))[None]
            if a == 0:
                qc_refs[0][pl.ds(k, 1)] = qc0[None]
            else:
                qc_refs[a][pl.ds(k, 1)] = jnp.where(
                    lane2 < _N - a, _rot_l(qc0, a), f32(0.0))[None]
        return (tuple(vs), (p1[1], p1[2], p1[3], p1[4]), kmj + 1)

    kmj0 = 1 - lane2
    lax.fori_loop(1, _ND, fwd_body, (p1, p2, kmj0), unroll=4)

    zeros_row = jnp.zeros((1, _B, _N), f32)
    for a in range(_A):
        qbs_refs[a][pl.ds(_ND, 1)] = zeros_row

    e_seed = jnp.where(lane2 == _N - 1, f32(1.0), f32(0.0))
    esk_ref[pl.ds(_ND - 1, 1)] = e_seed[None]

    def bwd_body(s, carry):
        q1, q2 = carry
        k = _ND - 2 - s
        es = []
        for a in range(_A):
            qas = qas_refs[a][pl.ds(k + 1, 1)][0]
            qbs = qbs_refs[a][pl.ds(k + 2, 1)][0]
            qc = qc_refs[a][pl.ds(k + 1, 1)][0]
            es.append((qas * q1[a + 1] + qbs * q2[a]) + qc * q1[a])
        e_top = jnp.where(lane2 >= _N - _A, f32(0.0), _rot_l(es[0], _A))
        es.append(e_top)
        esk_ref[pl.ds(k, 1)] = es[0][None]
        return (tuple(es), (q1[1], q1[2], q1[3], q1[4]))

    q1_init = tuple(
        jnp.where(lane2 == _N - 1 - a, f32(1.0), f32(0.0))
        for a in range(_A + 1))
    q2_init = tuple(jnp.zeros((_B, _N), f32) for _ in range(_A))
    lax.fori_loop(0, _ND - 1, bwd_body, (q1_init, q2_init), unroll=4)

    cur2 = jnp.mean(esk_ref[...], axis=1)
    lane2b = lax.broadcasted_iota(jnp.int32, (2 * _N, _N), 1)
    for t in range(7):
        s = 1 << t
        rolled = jnp.concatenate([cur2[s:], cur2[:s]], axis=0)
        cur2 = jnp.where((lane2b >> t) & 1 == 1, rolled, cur2)
    out_ref[...] = cur2[:_N]


@functools.partial(jax.jit, static_argnames=())
def _dtw_batch(D):
    Dt = jnp.transpose(D, (1, 0, 2))
    scratch = [pltpu.VMEM((_N, _B, _N), jnp.float32)] * _A
    scratch += [pltpu.VMEM((2 * _N, _B, _N), jnp.float32)] * (3 * _A + 1)
    return pl.pallas_call(
        _dtw_kernel,
        out_shape=jax.ShapeDtypeStruct((_N, _N), jnp.float32),
        in_specs=[pl.BlockSpec(memory_space=pltpu.VMEM)],
        out_specs=pl.BlockSpec(memory_space=pltpu.VMEM),
        scratch_shapes=scratch,
    )(Dt)


def kernel(D):
    return _dtw_batch(D)
